# baseline (speedup 1.0000x reference)
import functools

import jax
import jax.numpy as jnp
from jax import lax
from jax.experimental import pallas as pl
from jax.experimental.pallas import tpu as pltpu
from jax.experimental.pallas import tpu_sc as plsc

BATCH = 4
SEQ_LEN = 2048
D_MODEL = 768
_ROWS = BATCH * SEQ_LEN

_NUM_CORES = 2
_NUM_SUBCORES = 16
_NW = _NUM_CORES * _NUM_SUBCORES
_S_PER_W = SEQ_LEN // _NW
_HALF = _S_PER_W // 2
_NHC = BATCH * 2
_NPAIR = _NHC // 2
_LANES = 16
_D_SLICES = D_MODEL // _LANES


def _body(x_hbm, tok_hbm, pos_hbm, out_hbm, idx_v, tok_v, pos_v,
          gs0, gs1, ss0, ss1, psem):
    wid = lax.axis_index("s") * _NUM_CORES + lax.axis_index("c")
    s_base = wid * _S_PER_W
    gsems = (gs0, gs1)
    ssems = (ss0, ss1)

    for b in range(BATCH):
        pltpu.sync_copy(x_hbm.at[b, pl.ds(s_base, _S_PER_W)],
                        idx_v.at[pl.ds(b * _S_PER_W, _S_PER_W)])

    def gather(i, slot):
        return pltpu.make_async_copy(
            tok_hbm.at[idx_v.at[pl.ds(i * _HALF, _HALF)]],
            tok_v.at[pl.ds(slot * _HALF, _HALF)], gsems[slot])

    def store(i, slot):
        row_base = lax.div(i, 2) * SEQ_LEN + s_base + lax.rem(i, 2) * _HALF
        return pltpu.make_async_copy(
            tok_v.at[pl.ds(slot * _HALF, _HALF)],
            out_hbm.at[pl.ds(row_base, _HALF)], ssems[slot])

    def add_rows(i, slot):
        pos_off = lax.rem(i, 2) * _HALF
        buf_off = slot * _HALF

        def per_row(r, _):
            for j in range(_D_SLICES):
                sl = pl.ds(j * _LANES, _LANES)
                plsc.addupdate(tok_v.at[buf_off + r, sl],
                               pos_v[pos_off + r, sl])
            return 0

        lax.fori_loop(0, _HALF, per_row, 0, unroll=False)

    gather(0, 0).start()
    pos_cp = pltpu.make_async_copy(pos_hbm.at[pl.ds(s_base, _S_PER_W)],
                                   pos_v, psem)
    pos_cp.start()

    def pair(g, _):
        for k in range(2):
            i = 2 * g + k
            if k == 0:
                @pl.when(g > 0)
                def _():
                    store(i - 1, 1).wait()
                gather(i + 1, 1).start()
            else:
                store(i - 1, 0).wait()

                @pl.when(g < _NPAIR - 1)
                def _():
                    gather(i + 1, 0).start()
            gather(i, k).wait()
            if k == 0:
                @pl.when(g == 0)
                def _():
                    pos_cp.wait()
            add_rows(i, k)
            store(i, k).start()
        return 0

    lax.fori_loop(0, _NPAIR, pair, 0, unroll=False)
    store(_NHC - 1, 1).wait()


@functools.partial(
    pl.kernel,
    out_type=jax.ShapeDtypeStruct((_ROWS, D_MODEL), jnp.float32),
    mesh=plsc.VectorSubcoreMesh(core_axis_name="c", subcore_axis_name="s"),
    scratch_types=[
        pltpu.VMEM((_NHC * _HALF,), jnp.int32),
        pltpu.VMEM((_S_PER_W, D_MODEL), jnp.float32),
        pltpu.VMEM((_S_PER_W, D_MODEL), jnp.float32),
        pltpu.SemaphoreType.DMA,
        pltpu.SemaphoreType.DMA,
        pltpu.SemaphoreType.DMA,
        pltpu.SemaphoreType.DMA,
        pltpu.SemaphoreType.DMA,
    ],
)
def _emb_lookup(x_hbm, tok_hbm, pos_hbm, out_hbm, idx_v, tok_v, pos_v,
                gs0, gs1, ss0, ss1, psem):
    _body(x_hbm, tok_hbm, pos_hbm, out_hbm, idx_v, tok_v, pos_v,
          gs0, gs1, ss0, ss1, psem)


def kernel(x, token_table, position_table):
    x = x.astype(jnp.int32)
    out = _emb_lookup(x, token_table, position_table)
    return out.reshape(BATCH, SEQ_LEN, D_MODEL)
